# baseline (speedup 1.0000x reference)
import functools

import jax
import jax.numpy as jnp
from jax import lax
from jax.experimental import pallas as pl
from jax.experimental.pallas import tpu as pltpu
from jax.experimental.pallas import tpu_sc as plsc

N_WORKERS = 32
N_CHUNKS = 8


def _sc_gather(idx2, idx1, vm8, emb2w8, emb1w8, pairs_pw, D):
    rows_pc = idx2.shape[2]
    pairs_pc = rows_pc // D
    mesh = plsc.VectorSubcoreMesh(core_axis_name="c", subcore_axis_name="s")
    nc = mesh.num_cores
    L = 16

    def body(idx_hbm, idx1_hbm, vm_hbm, emb2_hbm, emb1_hbm,
             out2_hbm, out1_hbm,
             idx_v, rows_v, deep_v, idx1_v, e1_v, vm_v, e1s_v,
             sem, sem1):
        wid = lax.axis_index("s") * nc + lax.axis_index("c")
        iota = lax.iota(jnp.int32, L)
        pltpu.sync_copy(idx1_hbm.at[wid], idx1_v)
        pltpu.sync_copy(vm_hbm.at[wid], vm_v)
        cp1 = pltpu.async_copy(emb1_hbm.at[idx1_v], e1_v, sem1)
        for c in range(N_CHUNKS):
            pltpu.sync_copy(idx_hbm.at[wid, c], idx_v)
            pltpu.async_copy(emb2_hbm.at[idx_v], rows_v, sem).wait()

            def sel_body(g, carry):
                pvec = g * L + iota
                vm16 = vm_v[pl.ds(c * pairs_pc + g * L, L)]
                col = jnp.right_shift(vm16, 1)
                sh = jnp.bitwise_and(vm16, 1) * 16
                for d in range(D):
                    vals = plsc.load_gather(rows_v, [pvec * D + d, col])
                    bits = plsc.bitcast(vals, jnp.int32)
                    bits = jnp.left_shift(
                        jnp.bitwise_and(jnp.right_shift(bits, sh), 0xFFFF), 16)
                    plsc.store_scatter(deep_v, [pvec * D + d],
                                       plsc.bitcast(bits, jnp.float32))
                return carry

            lax.fori_loop(0, pairs_pc // L, sel_body, 0)
            pltpu.sync_copy(deep_v, out2_hbm.at[wid, c])
        cp1.wait()

        def sel1_body(g, carry):
            pvec = g * L + iota
            vm16 = vm_v[pl.ds(g * L, L)]
            col = jnp.right_shift(vm16, 1)
            sh = jnp.bitwise_and(vm16, 1) * 16
            vals = plsc.load_gather(e1_v, [pvec, col])
            bits = plsc.bitcast(vals, jnp.int32)
            bits = jnp.left_shift(
                jnp.bitwise_and(jnp.right_shift(bits, sh), 0xFFFF), 16)
            plsc.store_scatter(e1s_v, [pvec], plsc.bitcast(bits, jnp.float32))
            return carry

        lax.fori_loop(0, pairs_pw // L, sel1_body, 0)
        pltpu.sync_copy(e1s_v, out1_hbm.at[wid])

    call = pl.kernel(
        body,
        out_type=[
            jax.ShapeDtypeStruct((N_WORKERS, N_CHUNKS, rows_pc), jnp.float32),
            jax.ShapeDtypeStruct((N_WORKERS, pairs_pw), jnp.float32),
        ],
        mesh=mesh,
        scratch_types=[
            pltpu.VMEM((rows_pc,), jnp.int32),
            pltpu.VMEM((rows_pc, 8), jnp.float32),
            pltpu.VMEM((rows_pc,), jnp.float32),
            pltpu.VMEM((pairs_pw,), jnp.int32),
            pltpu.VMEM((pairs_pw, 8), jnp.float32),
            pltpu.VMEM((pairs_pw,), jnp.int32),
            pltpu.VMEM((pairs_pw,), jnp.float32),
            pltpu.SemaphoreType.DMA,
            pltpu.SemaphoreType.DMA,
        ],
        compiler_params=pltpu.CompilerParams(
            use_tc_tiling_on_sc=False, needs_layout_passes=False),
    )
    return call(idx2, idx1, vm8.reshape(N_WORKERS, pairs_pw), emb2w8, emb1w8)


def _tc_dense(raw, e1w, Xv, E, S, W1, b1, W2, b2, bias, blk):
    B, FD = raw.shape
    F = Xv.shape[1]
    hp = jax.lax.Precision.HIGHEST

    def body(xv_ref, e1_ref, raw_ref, E_ref, S_ref, W1_ref, b1_ref,
             W2_ref, b2_ref, bias_ref, out_ref):
        xv = xv_ref[...]
        rawb = raw_ref[...]
        xve = jnp.dot(xv, E_ref[...], precision=hp)
        deep = rawb * xve
        fm1 = jnp.sum(e1_ref[...] * xv, axis=1)
        se = jnp.dot(deep, S_ref[...], precision=hp)
        sq = jnp.dot(deep * deep, S_ref[...], precision=hp)
        fm2 = 0.5 * jnp.sum(se * se - sq, axis=1)
        x1 = jnp.maximum(jnp.dot(deep, W1_ref[...], precision=hp) + b1_ref[...], 0.0)
        x2 = jnp.maximum(jnp.dot(x1, W2_ref[...], precision=hp) + b2_ref[...], 0.0)
        out_ref[...] = fm1 + fm2 + jnp.sum(x2, axis=1) + bias_ref[0, 0]

    grid = B // blk
    full = lambda shape: pl.BlockSpec(shape, lambda i: (0,) * len(shape))
    return pl.pallas_call(
        body,
        grid=(grid,),
        in_specs=[
            pl.BlockSpec((blk, F), lambda i: (i, 0)),
            pl.BlockSpec((blk, F), lambda i: (i, 0)),
            pl.BlockSpec((blk, FD), lambda i: (i, 0)),
            full(E.shape),
            full(S.shape),
            full(W1.shape),
            full((b1.shape[0],)),
            full(W2.shape),
            full((b2.shape[0],)),
            pl.BlockSpec(memory_space=pltpu.SMEM),
        ],
        out_specs=pl.BlockSpec((blk,), lambda i: (i,)),
        out_shape=jax.ShapeDtypeStruct((B,), jnp.float32),
        compiler_params=pltpu.CompilerParams(
            dimension_semantics=("arbitrary",),
        ),
    )(Xv, e1w, raw, E, S, W1, b1, W2, b2, bias.reshape(1, 1))


def kernel(Xi, Xv, emb1, emb2, W1, b1, W2, b2, bias):
    F, V, D = emb2.shape
    B = Xi.shape[0]
    v = Xi[..., 0].astype(jnp.int32)
    v16 = jnp.right_shift(v, 4)
    vm16 = jnp.bitwise_and(v, 15)
    fd = (jnp.arange(F, dtype=jnp.int32) * D)[None, :, None] + \
        jnp.arange(D, dtype=jnp.int32)[None, None, :]
    pairs_pc = (B * F) // (N_WORKERS * N_CHUNKS)
    idx2 = (fd * (V // 16) + v16[:, :, None]).reshape(
        N_WORKERS, N_CHUNKS, pairs_pc * D)
    idx1 = ((jnp.arange(F, dtype=jnp.int32) * (V // 16))[None, :] + v16).reshape(
        N_WORKERS, (B * F) // N_WORKERS)

    emb2w8 = jax.lax.bitcast_convert_type(
        jnp.transpose(emb2, (0, 2, 1)).astype(jnp.bfloat16).reshape(
            F * D * V // 16, 8, 2),
        jnp.float32)
    emb1w8 = jax.lax.bitcast_convert_type(
        emb1.astype(jnp.bfloat16).reshape(F * V // 16, 8, 2), jnp.float32)
    e2sel, e1sel = _sc_gather(idx2, idx1, vm16, emb2w8, emb1w8,
                              (B * F) // N_WORKERS, D)

    raw = e2sel.reshape(B, F * D)
    e1w = e1sel.reshape(B, F)

    E = jnp.kron(jnp.eye(F, dtype=jnp.float32), jnp.ones((1, D), jnp.float32))
    S = jnp.tile(jnp.eye(D, dtype=jnp.float32), (F, 1))

    return _tc_dense(raw, e1w, Xv, E, S, W1, b1, W2, b2, bias, blk=512)


# speedup vs baseline: 101.0776x; 101.0776x over previous
import functools

import jax
import jax.numpy as jnp
from jax import lax
from jax.experimental import pallas as pl
from jax.experimental.pallas import tpu as pltpu
from jax.experimental.pallas import tpu_sc as plsc

N_WORKERS = 32
N_CHUNKS = 8


def _sc_gather(idx2, idx1, vm8, emb2w8, emb1w8, pairs_pw, D):
    rows_pc = idx2.shape[2]
    pairs_pc = rows_pc // D
    mesh = plsc.VectorSubcoreMesh(core_axis_name="c", subcore_axis_name="s")
    nc = mesh.num_cores
    L = 16

    def body(idx_hbm, idx1_hbm, vm_hbm, emb2_hbm, emb1_hbm,
             out2_hbm, out1_hbm,
             idx_v, rows_v, deep_v, idx1_v, e1_v, vm_v, e1s_v,
             sem, sem1):
        wid = lax.axis_index("s") * nc + lax.axis_index("c")
        iota = lax.iota(jnp.int32, L)
        pltpu.sync_copy(idx1_hbm.at[wid], idx1_v)
        pltpu.sync_copy(vm_hbm.at[wid], vm_v)
        cp1 = pltpu.async_copy(emb1_hbm.at[idx1_v], e1_v, sem1)
        for c in range(N_CHUNKS):
            pltpu.sync_copy(idx_hbm.at[wid, c], idx_v)
            pltpu.async_copy(emb2_hbm.at[idx_v], rows_v, sem).wait()

            def sel_body(g, carry):
                pvec = g * L + iota
                lane = vm_v[pl.ds(c * pairs_pc + g * L, L)]
                for d in range(D):
                    vals = plsc.load_gather(rows_v, [pvec * D + d, lane])
                    plsc.store_scatter(deep_v, [pvec * D + d], vals)
                return carry

            lax.fori_loop(0, pairs_pc // L, sel_body, 0)
            pltpu.sync_copy(deep_v, out2_hbm.at[wid, c])
        cp1.wait()

        def sel1_body(g, carry):
            pvec = g * L + iota
            lane = vm_v[pl.ds(g * L, L)]
            vals = plsc.load_gather(e1_v, [pvec, lane])
            plsc.store_scatter(e1s_v, [pvec], vals)
            return carry

        lax.fori_loop(0, pairs_pw // L, sel1_body, 0)
        pltpu.sync_copy(e1s_v, out1_hbm.at[wid])

    call = pl.kernel(
        body,
        out_type=[
            jax.ShapeDtypeStruct((N_WORKERS, N_CHUNKS, rows_pc), jnp.float32),
            jax.ShapeDtypeStruct((N_WORKERS, pairs_pw), jnp.float32),
        ],
        mesh=mesh,
        scratch_types=[
            pltpu.VMEM((rows_pc,), jnp.int32),
            pltpu.VMEM((rows_pc, 8), jnp.float32),
            pltpu.VMEM((rows_pc,), jnp.float32),
            pltpu.VMEM((pairs_pw,), jnp.int32),
            pltpu.VMEM((pairs_pw, 8), jnp.float32),
            pltpu.VMEM((pairs_pw,), jnp.int32),
            pltpu.VMEM((pairs_pw,), jnp.float32),
            pltpu.SemaphoreType.DMA,
            pltpu.SemaphoreType.DMA,
        ],
        compiler_params=pltpu.CompilerParams(
            use_tc_tiling_on_sc=False, needs_layout_passes=False),
    )
    return call(idx2, idx1, vm8.reshape(N_WORKERS, pairs_pw), emb2w8, emb1w8)


def _tc_dense(raw, e1w, Xv, E, S, W1, b1, W2, b2, bias, blk):
    B, FD = raw.shape
    F = Xv.shape[1]
    hp = jax.lax.Precision.HIGHEST

    def body(xv_ref, e1_ref, raw_ref, E_ref, S_ref, W1_ref, b1_ref,
             W2_ref, b2_ref, bias_ref, out_ref):
        xv = xv_ref[...]
        rawb = raw_ref[...]
        xve = jnp.dot(xv, E_ref[...], precision=hp)
        deep = rawb * xve
        fm1 = jnp.sum(e1_ref[...] * xv, axis=1)
        se = jnp.dot(deep, S_ref[...], precision=hp)
        sq = jnp.dot(deep * deep, S_ref[...], precision=hp)
        fm2 = 0.5 * jnp.sum(se * se - sq, axis=1)
        x1 = jnp.maximum(jnp.dot(deep, W1_ref[...], precision=hp) + b1_ref[...], 0.0)
        x2 = jnp.maximum(jnp.dot(x1, W2_ref[...], precision=hp) + b2_ref[...], 0.0)
        out_ref[...] = fm1 + fm2 + jnp.sum(x2, axis=1) + bias_ref[0, 0]

    grid = B // blk
    full = lambda shape: pl.BlockSpec(shape, lambda i: (0,) * len(shape))
    return pl.pallas_call(
        body,
        grid=(grid,),
        in_specs=[
            pl.BlockSpec((blk, F), lambda i: (i, 0)),
            pl.BlockSpec((blk, F), lambda i: (i, 0)),
            pl.BlockSpec((blk, FD), lambda i: (i, 0)),
            full(E.shape),
            full(S.shape),
            full(W1.shape),
            full((b1.shape[0],)),
            full(W2.shape),
            full((b2.shape[0],)),
            pl.BlockSpec(memory_space=pltpu.SMEM),
        ],
        out_specs=pl.BlockSpec((blk,), lambda i: (i,)),
        out_shape=jax.ShapeDtypeStruct((B,), jnp.float32),
        compiler_params=pltpu.CompilerParams(
            dimension_semantics=("arbitrary",),
        ),
    )(Xv, e1w, raw, E, S, W1, b1, W2, b2, bias.reshape(1, 1))


def kernel(Xi, Xv, emb1, emb2, W1, b1, W2, b2, bias):
    F, V, D = emb2.shape
    B = Xi.shape[0]
    v = Xi[..., 0].astype(jnp.int32)
    v8 = jnp.right_shift(v, 3)
    vm8 = jnp.bitwise_and(v, 7)
    fd = (jnp.arange(F, dtype=jnp.int32) * D)[None, :, None] + \
        jnp.arange(D, dtype=jnp.int32)[None, None, :]
    pairs_pc = (B * F) // (N_WORKERS * N_CHUNKS)
    idx2 = (fd * (V // 8) + v8[:, :, None]).reshape(
        N_WORKERS, N_CHUNKS, pairs_pc * D)
    idx1 = ((jnp.arange(F, dtype=jnp.int32) * (V // 8))[None, :] + v8).reshape(
        N_WORKERS, (B * F) // N_WORKERS)

    emb2w8 = jnp.transpose(emb2, (0, 2, 1)).reshape(F * D * V // 8, 8)
    emb1w8 = emb1.reshape(F * V // 8, 8)
    e2sel, e1sel = _sc_gather(idx2, idx1, vm8, emb2w8, emb1w8,
                              (B * F) // N_WORKERS, D)

    raw = e2sel.reshape(B, F * D)
    e1w = e1sel.reshape(B, F)

    E = jnp.kron(jnp.eye(F, dtype=jnp.float32), jnp.ones((1, D), jnp.float32))
    S = jnp.tile(jnp.eye(D, dtype=jnp.float32), (F, 1))

    return _tc_dense(raw, e1w, Xv, E, S, W1, b1, W2, b2, bias, blk=512)
